# Initial kernel scaffold; baseline (speedup 1.0000x reference)
import jax
import jax.numpy as jnp
from jax.experimental import pallas as pl


def kernel(imgs, masks, msgs, w_img_t, w_msg, b_h, w_out_t, b_out, w_det_t, b_det):
    raise NotImplementedError("write your pallas kernel here")


# speedup vs baseline: 1.7261x; 1.7261x over previous
import functools

import jax
import jax.numpy as jnp
from jax import lax
from jax.experimental import pallas as pl
from jax.experimental.pallas import tpu as pltpu

_NBITS = 8
_HIDDEN = 32
_PRED_CH = 1 + _NBITS
_MAX_TILE = 16384
_CHUNK = 1024
_OUT_DTYPE = jnp.bfloat16


def _wam_kernel(imgs_ref, mask_ref, mbias_ref,
                wimg_ref, wout_ref, bout_ref, wdet_ref, bdet_ref,
                imgs_w_ref, comb_ref, preds_ref, *, chunk):
    mbias = mbias_ref[0]
    wimg = wimg_ref[...]
    wout = wout_ref[...]
    bout = bout_ref[...]
    wdet = wdet_ref[...]
    bdet = bdet_ref[...]

    n_chunks = imgs_ref.shape[2] // chunk

    def body(c, carry):
        off = pl.multiple_of(c * chunk, 128)
        x = imgs_ref[0, :, pl.ds(off, chunk)]
        m = mask_ref[0, :, pl.ds(off, chunk)]

        hpre = jnp.dot(wimg, x, preferred_element_type=jnp.float32) + mbias
        h = jnp.tanh(hpre.astype(jnp.bfloat16)).astype(jnp.float32)
        delta = jnp.tanh(
            jnp.dot(wout, h, preferred_element_type=jnp.float32) + bout)

        imgs_w = x + delta
        combined = x + m * delta
        preds = (jnp.dot(wdet, combined, preferred_element_type=jnp.float32)
                 + bdet)

        imgs_w_ref[0, :, pl.ds(off, chunk)] = imgs_w.astype(imgs_w_ref.dtype)
        comb_ref[0, :, pl.ds(off, chunk)] = combined.astype(comb_ref.dtype)
        preds_ref[0, :, pl.ds(off, chunk)] = preds.astype(preds_ref.dtype)
        return carry

    lax.fori_loop(0, n_chunks, body, 0, unroll=False)


def _pick_hw_tile(hw):
    t = pl.cdiv(hw, _CHUNK) * _CHUNK
    return min(t, _MAX_TILE)


def _wam_fused(imgs_flat, mask_flat, mbias,
               w_img_t, w_out_t, b_out, w_det_t, b_det):
    B, C, HWp = imgs_flat.shape
    T = _pick_hw_tile(HWp)
    assert HWp % T == 0
    n_t = HWp // T

    kernel_fn = functools.partial(_wam_kernel, chunk=min(_CHUNK, T))

    def img_map(i):
        return (i // n_t, 0, i % n_t)

    def per_batch_map(i):
        return (i // n_t, 0, 0)

    def weight_map(i):
        return (0, 0)

    in_specs = [
        pl.BlockSpec((1, C, T), img_map),
        pl.BlockSpec((1, 1, T), img_map),
        pl.BlockSpec((1, _HIDDEN, 1), per_batch_map),
        pl.BlockSpec(w_img_t.shape, weight_map),
        pl.BlockSpec(w_out_t.shape, weight_map),
        pl.BlockSpec(b_out.shape, weight_map),
        pl.BlockSpec(w_det_t.shape, weight_map),
        pl.BlockSpec(b_det.shape, weight_map),
    ]
    out_specs = (
        pl.BlockSpec((1, C, T), img_map),
        pl.BlockSpec((1, C, T), img_map),
        pl.BlockSpec((1, _PRED_CH, T), img_map),
    )
    out_shapes = (
        jax.ShapeDtypeStruct((B, C, HWp), _OUT_DTYPE),
        jax.ShapeDtypeStruct((B, C, HWp), _OUT_DTYPE),
        jax.ShapeDtypeStruct((B, _PRED_CH, HWp), _OUT_DTYPE),
    )

    return pl.pallas_call(
        kernel_fn,
        out_shape=out_shapes,
        grid_spec=pltpu.PrefetchScalarGridSpec(
            num_scalar_prefetch=0,
            grid=(B * n_t,),
            in_specs=in_specs,
            out_specs=out_specs),
        compiler_params=pltpu.CompilerParams(
            dimension_semantics=("parallel",)),
    )(imgs_flat, mask_flat, mbias,
      w_img_t, w_out_t, b_out, w_det_t, b_det)


def kernel(imgs, masks, msgs, w_img_t, w_msg, b_h, w_out_t, b_out,
           w_det_t, b_det):
    B, C, H, W = imgs.shape
    HW = H * W

    imgs_flat = imgs.reshape(B, C, HW)
    mask_flat = masks.reshape(B, 1, HW)

    T = _pick_hw_tile(HW)
    HWp = pl.cdiv(HW, T) * T
    if HWp != HW:
        pad = HWp - HW
        imgs_flat = jnp.pad(imgs_flat, ((0, 0), (0, 0), (0, pad)))
        mask_flat = jnp.pad(mask_flat, ((0, 0), (0, 0), (0, pad)))

    msg_pm1 = 2.0 * msgs.astype(jnp.float32) - 1.0
    mbias = (msg_pm1 @ w_msg + b_h).reshape(B, _HIDDEN, 1)

    imgs_w_flat, comb_flat, preds_flat = _wam_fused(
        imgs_flat, mask_flat, mbias,
        w_img_t, w_out_t, b_out, w_det_t, b_det)

    return (imgs_w_flat[:, :, :HW].reshape(B, C, H, W),
            comb_flat[:, :, :HW].reshape(B, C, H, W),
            preds_flat[:, :, :HW].reshape(B, _PRED_CH, H, W))


# speedup vs baseline: 5.0583x; 2.9304x over previous
import functools

import jax
import jax.numpy as jnp
from jax import lax
from jax.experimental import pallas as pl
from jax.experimental.pallas import tpu as pltpu

_NBITS = 8
_HIDDEN = 32
_PRED_CH = 1 + _NBITS
_MAX_TILE = 16384
_CHUNK = 8192
_OUT_DTYPE = jnp.bfloat16


def _wam_kernel(imgs_ref, mask_ref, mbias_ref,
                wimg_ref, wout_ref, bout_ref, wdet_ref, bdet_ref,
                imgs_w_ref, comb_ref, preds_ref, *, chunk):
    mbias = mbias_ref[0]
    wimg = wimg_ref[...]
    wout = wout_ref[...].astype(jnp.bfloat16)
    bout = bout_ref[...]
    wdet = wdet_ref[...].astype(jnp.bfloat16)
    bdet = bdet_ref[...]

    n_chunks = imgs_ref.shape[2] // chunk

    def body(c, carry):
        off = pl.multiple_of(c * chunk, 128)
        x = imgs_ref[0, :, pl.ds(off, chunk)]
        m = mask_ref[0, :, pl.ds(off, chunk)]

        hpre = jnp.dot(wimg, x, preferred_element_type=jnp.float32) + mbias
        h = jnp.tanh(hpre.astype(jnp.bfloat16))
        delta = jnp.tanh(
            jnp.dot(wout, h, preferred_element_type=jnp.float32) + bout)

        imgs_w = x + delta
        combined = (x + m * delta).astype(jnp.bfloat16)
        preds = (jnp.dot(wdet, combined, preferred_element_type=jnp.float32)
                 + bdet)

        imgs_w_ref[0, :, pl.ds(off, chunk)] = imgs_w.astype(imgs_w_ref.dtype)
        comb_ref[0, :, pl.ds(off, chunk)] = combined
        preds_ref[0, :, pl.ds(off, chunk)] = preds.astype(preds_ref.dtype)
        return carry

    lax.fori_loop(0, n_chunks, body, 0, unroll=2)


def _pick_hw_tile(hw):
    t = pl.cdiv(hw, _CHUNK) * _CHUNK
    return min(t, _MAX_TILE)


def _wam_fused(imgs_flat, mask_flat, mbias,
               w_img_t, w_out_t, b_out, w_det_t, b_det):
    B, C, HWp = imgs_flat.shape
    T = _pick_hw_tile(HWp)
    assert HWp % T == 0
    n_t = HWp // T

    kernel_fn = functools.partial(_wam_kernel, chunk=min(_CHUNK, T))

    def img_map(i):
        return (i // n_t, 0, i % n_t)

    def per_batch_map(i):
        return (i // n_t, 0, 0)

    def weight_map(i):
        return (0, 0)

    in_specs = [
        pl.BlockSpec((1, C, T), img_map),
        pl.BlockSpec((1, 1, T), img_map),
        pl.BlockSpec((1, _HIDDEN, 1), per_batch_map),
        pl.BlockSpec(w_img_t.shape, weight_map),
        pl.BlockSpec(w_out_t.shape, weight_map),
        pl.BlockSpec(b_out.shape, weight_map),
        pl.BlockSpec(w_det_t.shape, weight_map),
        pl.BlockSpec(b_det.shape, weight_map),
    ]
    out_specs = (
        pl.BlockSpec((1, C, T), img_map),
        pl.BlockSpec((1, C, T), img_map),
        pl.BlockSpec((1, _PRED_CH, T), img_map),
    )
    out_shapes = (
        jax.ShapeDtypeStruct((B, C, HWp), _OUT_DTYPE),
        jax.ShapeDtypeStruct((B, C, HWp), _OUT_DTYPE),
        jax.ShapeDtypeStruct((B, _PRED_CH, HWp), _OUT_DTYPE),
    )

    return pl.pallas_call(
        kernel_fn,
        out_shape=out_shapes,
        grid_spec=pltpu.PrefetchScalarGridSpec(
            num_scalar_prefetch=0,
            grid=(B * n_t,),
            in_specs=in_specs,
            out_specs=out_specs),
        compiler_params=pltpu.CompilerParams(
            dimension_semantics=("parallel",)),
    )(imgs_flat, mask_flat, mbias,
      w_img_t, w_out_t, b_out, w_det_t, b_det)


def kernel(imgs, masks, msgs, w_img_t, w_msg, b_h, w_out_t, b_out,
           w_det_t, b_det):
    B, C, H, W = imgs.shape
    HW = H * W

    imgs_flat = imgs.reshape(B, C, HW)
    mask_flat = masks.reshape(B, 1, HW)

    T = _pick_hw_tile(HW)
    HWp = pl.cdiv(HW, T) * T
    if HWp != HW:
        pad = HWp - HW
        imgs_flat = jnp.pad(imgs_flat, ((0, 0), (0, 0), (0, pad)))
        mask_flat = jnp.pad(mask_flat, ((0, 0), (0, 0), (0, pad)))

    msg_pm1 = 2.0 * msgs.astype(jnp.float32) - 1.0
    mbias = (msg_pm1 @ w_msg + b_h).reshape(B, _HIDDEN, 1)

    imgs_w_flat, comb_flat, preds_flat = _wam_fused(
        imgs_flat, mask_flat, mbias,
        w_img_t, w_out_t, b_out, w_det_t, b_det)

    return (imgs_w_flat[:, :, :HW].reshape(B, C, H, W),
            comb_flat[:, :, :HW].reshape(B, C, H, W),
            preds_flat[:, :, :HW].reshape(B, _PRED_CH, H, W))


# speedup vs baseline: 7.7788x; 1.5378x over previous
import functools

import jax
import jax.numpy as jnp
from jax import lax
from jax.experimental import pallas as pl
from jax.experimental.pallas import tpu as pltpu

_NBITS = 8
_HIDDEN = 32
_PRED_CH = 1 + _NBITS
_ROWS_PER_STEP = 64
_ROWS_PER_CHUNK = 8


def _tree_sum(xs):
    while len(xs) > 1:
        nxt = [a + b for a, b in zip(xs[::2], xs[1::2])]
        if len(xs) % 2:
            nxt.append(xs[-1])
        xs = nxt
    return xs[0]


def _wam_kernel(mbias_ref, wimg_ref, wout_ref, bout_ref, wdet_ref, bdet_ref,
                imgs_ref, mask_ref,
                imgs_w_ref, comb_ref, preds_ref, *, hc, n_rt):
    b = pl.program_id(0) // n_rt
    C = imgs_ref.shape[1]
    hb = imgs_ref.shape[2]
    n_chunks = hb // hc

    wimg = [[wimg_ref[k, c] for c in range(C)] for k in range(_HIDDEN)]
    mbias = [mbias_ref[b, k] for k in range(_HIDDEN)]
    wout = [[wout_ref[c, k] for k in range(_HIDDEN)] for c in range(C)]
    bout = [bout_ref[c, 0] for c in range(C)]
    wdet = [[wdet_ref[p, c] for c in range(C)] for p in range(_PRED_CH)]
    bdet = [bdet_ref[p, 0] for p in range(_PRED_CH)]

    def body(r, carry):
        r0 = pl.multiple_of(r * hc, hc)
        x = [imgs_ref[0, c, pl.ds(r0, hc), :] for c in range(C)]
        m = mask_ref[0, 0, pl.ds(r0, hc), :]

        hs = []
        for k in range(_HIDDEN):
            a = _tree_sum([x[c] * wimg[k][c] for c in range(C)]) + mbias[k]
            hs.append(jnp.tanh(a.astype(jnp.bfloat16)).astype(jnp.float32))

        deltas = []
        for c in range(C):
            acc = _tree_sum([hs[k] * wout[c][k] for k in range(_HIDDEN)])
            deltas.append(jnp.tanh(acc + bout[c]))

        combs = []
        for c in range(C):
            imgs_w_ref[0, c, pl.ds(r0, hc), :] = (
                (x[c] + deltas[c]).astype(imgs_w_ref.dtype))
            cb = x[c] + m * deltas[c]
            comb_ref[0, c, pl.ds(r0, hc), :] = cb.astype(comb_ref.dtype)
            combs.append(cb)

        for p in range(_PRED_CH):
            pr = _tree_sum([combs[c] * wdet[p][c] for c in range(C)]) + bdet[p]
            preds_ref[0, p, pl.ds(r0, hc), :] = pr.astype(preds_ref.dtype)
        return carry

    lax.fori_loop(0, n_chunks, body, 0, unroll=2)


def _pick_rows(h):
    for hb in (_ROWS_PER_STEP, 32, 16, 8):
        if h % hb == 0:
            return hb
    return h


def kernel(imgs, masks, msgs, w_img_t, w_msg, b_h, w_out_t, b_out,
           w_det_t, b_det):
    B, C, H, W = imgs.shape
    hb = _pick_rows(H)
    n_rt = H // hb
    hc = _ROWS_PER_CHUNK if hb % _ROWS_PER_CHUNK == 0 else hb

    msg_pm1 = 2.0 * msgs.astype(jnp.float32) - 1.0
    mbias = msg_pm1 @ w_msg + b_h

    kernel_fn = functools.partial(_wam_kernel, hc=hc, n_rt=n_rt)

    def band_map(i, *_):
        return (i // n_rt, 0, i % n_rt, 0)

    in_specs = [
        pl.BlockSpec((1, C, hb, W), band_map),
        pl.BlockSpec((1, 1, hb, W), band_map),
    ]
    out_specs = (
        pl.BlockSpec((1, C, hb, W), band_map),
        pl.BlockSpec((1, C, hb, W), band_map),
        pl.BlockSpec((1, _PRED_CH, hb, W), band_map),
    )
    out_shapes = (
        jax.ShapeDtypeStruct((B, C, H, W), jnp.bfloat16),
        jax.ShapeDtypeStruct((B, C, H, W), jnp.bfloat16),
        jax.ShapeDtypeStruct((B, _PRED_CH, H, W), jnp.bfloat16),
    )

    return pl.pallas_call(
        kernel_fn,
        out_shape=out_shapes,
        grid_spec=pltpu.PrefetchScalarGridSpec(
            num_scalar_prefetch=6,
            grid=(B * n_rt,),
            in_specs=in_specs,
            out_specs=out_specs),
        compiler_params=pltpu.CompilerParams(
            dimension_semantics=("parallel",)),
    )(mbias, w_img_t, w_out_t, b_out, w_det_t, b_det, imgs, masks)


# speedup vs baseline: 11.3644x; 1.4609x over previous
import jax
import jax.numpy as jnp
from jax import lax
from jax.experimental import pallas as pl
from jax.experimental.pallas import tpu as pltpu

_NBITS = 8
_HIDDEN = 32
_PRED_CH = 1 + _NBITS
_ROWS_PER_STEP = 64


def _wam_kernel(imgs_ref, mask_ref, mbias_ref, wimg_ref, wout_ref, bout_ref,
                wdet_ref, bdet_ref, imgs_w_ref, comb_ref, preds_ref):
    C = imgs_ref.shape[1]
    hb = imgs_ref.shape[2]
    W = imgs_ref.shape[3]
    P = hb * W

    mbias = mbias_ref[0]
    wimg = wimg_ref[...]
    wout = wout_ref[...].astype(jnp.bfloat16)
    bout = bout_ref[...]
    wdet = wdet_ref[...].astype(jnp.bfloat16)
    bdet = bdet_ref[...]

    x = imgs_ref[0].reshape(C, P)
    m = mask_ref[0].reshape(1, P)

    hpre = jnp.dot(wimg, x, preferred_element_type=jnp.float32) + mbias
    h = jnp.tanh(hpre.astype(jnp.bfloat16))
    delta = jnp.tanh(
        jnp.dot(wout, h, preferred_element_type=jnp.float32) + bout)

    iw = (x + delta).astype(jnp.bfloat16)
    cb = (x + m * delta).astype(jnp.bfloat16)
    preds = (jnp.dot(wdet, cb, preferred_element_type=jnp.float32)
             + bdet).astype(jnp.bfloat16)

    imgs_w_ref[0] = iw.reshape(C, hb, W)
    comb_ref[0] = cb.reshape(C, hb, W)
    preds_ref[0] = preds.reshape(_PRED_CH, hb, W)


def _pick_rows(h):
    for hb in (_ROWS_PER_STEP, 32, 16, 8):
        if h % hb == 0:
            return hb
    return h


def kernel(imgs, masks, msgs, w_img_t, w_msg, b_h, w_out_t, b_out,
           w_det_t, b_det):
    B, C, H, W = imgs.shape
    hb = _pick_rows(H)
    n_rt = H // hb

    msg_pm1 = 2.0 * msgs.astype(jnp.float32) - 1.0
    mbias = (msg_pm1 @ w_msg + b_h).reshape(B, _HIDDEN, 1)

    def band_map(i):
        return (i // n_rt, 0, i % n_rt, 0)

    def batch_map(i):
        return (i // n_rt, 0, 0)

    def weight_map(i):
        return (0, 0)

    in_specs = [
        pl.BlockSpec((1, C, hb, W), band_map),
        pl.BlockSpec((1, 1, hb, W), band_map),
        pl.BlockSpec((1, _HIDDEN, 1), batch_map),
        pl.BlockSpec(w_img_t.shape, weight_map),
        pl.BlockSpec(w_out_t.shape, weight_map),
        pl.BlockSpec(b_out.shape, weight_map),
        pl.BlockSpec(w_det_t.shape, weight_map),
        pl.BlockSpec(b_det.shape, weight_map),
    ]
    out_specs = (
        pl.BlockSpec((1, C, hb, W), band_map),
        pl.BlockSpec((1, C, hb, W), band_map),
        pl.BlockSpec((1, _PRED_CH, hb, W), band_map),
    )
    out_shapes = (
        jax.ShapeDtypeStruct((B, C, H, W), jnp.bfloat16),
        jax.ShapeDtypeStruct((B, C, H, W), jnp.bfloat16),
        jax.ShapeDtypeStruct((B, _PRED_CH, H, W), jnp.bfloat16),
    )

    return pl.pallas_call(
        _wam_kernel,
        out_shape=out_shapes,
        grid_spec=pltpu.PrefetchScalarGridSpec(
            num_scalar_prefetch=0,
            grid=(B * n_rt,),
            in_specs=in_specs,
            out_specs=out_specs),
        compiler_params=pltpu.CompilerParams(
            dimension_semantics=("parallel",)),
    )(imgs, masks, mbias, w_img_t, w_out_t, b_out, w_det_t, b_det)


# speedup vs baseline: 11.5481x; 1.0162x over previous
import jax
import jax.numpy as jnp
from jax import lax
from jax.experimental import pallas as pl
from jax.experimental.pallas import tpu as pltpu

_NBITS = 8
_HIDDEN = 32
_PRED_CH = 1 + _NBITS
_ROWS_PER_STEP = 64


def _wam_kernel(imgs_ref, mask_ref, wimg_ref, wout_ref, bout_ref,
                wdet_ref, bdet_ref, imgs_w_ref, comb_ref, preds_ref):
    C = imgs_ref.shape[1]
    hb = imgs_ref.shape[2]
    W = imgs_ref.shape[3]
    P = hb * W

    wimg = wimg_ref[0]
    wout = wout_ref[...].astype(jnp.bfloat16)
    bout = bout_ref[...]
    wdet = wdet_ref[...].astype(jnp.bfloat16)
    bdet = bdet_ref[...]

    x = imgs_ref[0].reshape(C, P)
    m = mask_ref[0].reshape(1, P)
    xa = jnp.concatenate([x, jnp.ones((1, P), jnp.float32)], axis=0)

    hpre = jnp.dot(wimg, xa, preferred_element_type=jnp.float32)
    h = jnp.tanh(hpre.astype(jnp.bfloat16))
    delta = jnp.tanh(
        jnp.dot(wout, h, preferred_element_type=jnp.float32) + bout)

    iw = (x + delta).astype(jnp.bfloat16)
    cb = (x + m * delta).astype(jnp.bfloat16)
    preds = (jnp.dot(wdet, cb, preferred_element_type=jnp.float32)
             + bdet).astype(jnp.bfloat16)

    imgs_w_ref[0] = iw.reshape(C, hb, W)
    comb_ref[0] = cb.reshape(C, hb, W)
    preds_ref[0] = preds.reshape(_PRED_CH, hb, W)


def _pick_rows(h):
    for hb in (_ROWS_PER_STEP, 32, 16, 8):
        if h % hb == 0:
            return hb
    return h


def kernel(imgs, masks, msgs, w_img_t, w_msg, b_h, w_out_t, b_out,
           w_det_t, b_det):
    B, C, H, W = imgs.shape
    hb = _pick_rows(H)
    n_rt = H // hb

    msg_pm1 = 2.0 * msgs.astype(jnp.float32) - 1.0
    mbias = (msg_pm1 @ w_msg + b_h).reshape(B, _HIDDEN, 1)
    wimg_aug = jnp.concatenate(
        [jnp.broadcast_to(w_img_t.astype(jnp.float32)[None],
                          (B, _HIDDEN, C)), mbias], axis=2)

    def band_map(i):
        return (i // n_rt, 0, i % n_rt, 0)

    def batch_map(i):
        return (i // n_rt, 0, 0)

    def weight_map(i):
        return (0, 0)

    in_specs = [
        pl.BlockSpec((1, C, hb, W), band_map),
        pl.BlockSpec((1, 1, hb, W), band_map),
        pl.BlockSpec((1, _HIDDEN, C + 1), batch_map),
        pl.BlockSpec(w_out_t.shape, weight_map),
        pl.BlockSpec(b_out.shape, weight_map),
        pl.BlockSpec(w_det_t.shape, weight_map),
        pl.BlockSpec(b_det.shape, weight_map),
    ]
    out_specs = (
        pl.BlockSpec((1, C, hb, W), band_map),
        pl.BlockSpec((1, C, hb, W), band_map),
        pl.BlockSpec((1, _PRED_CH, hb, W), band_map),
    )
    out_shapes = (
        jax.ShapeDtypeStruct((B, C, H, W), jnp.bfloat16),
        jax.ShapeDtypeStruct((B, C, H, W), jnp.bfloat16),
        jax.ShapeDtypeStruct((B, _PRED_CH, H, W), jnp.bfloat16),
    )

    return pl.pallas_call(
        _wam_kernel,
        out_shape=out_shapes,
        grid_spec=pltpu.PrefetchScalarGridSpec(
            num_scalar_prefetch=0,
            grid=(B * n_rt,),
            in_specs=in_specs,
            out_specs=out_specs),
        compiler_params=pltpu.CompilerParams(
            dimension_semantics=("parallel",)),
    )(imgs, masks, wimg_aug, w_out_t, b_out, w_det_t, b_det)


# speedup vs baseline: 13.2797x; 1.1499x over previous
import jax
import jax.numpy as jnp
from jax import lax
from jax.experimental import pallas as pl
from jax.experimental.pallas import tpu as pltpu

_NBITS = 8
_HIDDEN = 32
_PRED_CH = 1 + _NBITS
_ROWS_PER_STEP = 128


def _wam_kernel(imgs_ref, mask_ref, wimg_ref, wout_ref, bout_ref,
                wdet_ref, bdet_ref, imgs_w_ref, comb_ref, preds_ref):
    C = imgs_ref.shape[1]
    hb = imgs_ref.shape[2]
    W = imgs_ref.shape[3]
    P = hb * W

    wimg = wimg_ref[0]
    wout = wout_ref[...].astype(jnp.bfloat16)
    bout = bout_ref[...]
    wdet = wdet_ref[...].astype(jnp.bfloat16)
    bdet = bdet_ref[...]

    x = imgs_ref[0].reshape(C, P)
    m = mask_ref[0].reshape(1, P)
    xa = jnp.concatenate([x, jnp.ones((1, P), jnp.float32)], axis=0)

    hpre = jnp.dot(wimg, xa, preferred_element_type=jnp.float32)
    h = jnp.tanh(hpre.astype(jnp.bfloat16))
    delta = jnp.tanh(
        jnp.dot(wout, h, preferred_element_type=jnp.float32) + bout)

    iw = (x + delta).astype(jnp.bfloat16)
    cb = (x + m * delta).astype(jnp.bfloat16)
    preds = (jnp.dot(wdet, cb, preferred_element_type=jnp.float32)
             + bdet).astype(jnp.bfloat16)

    imgs_w_ref[0] = iw.reshape(C, hb, W)
    comb_ref[0] = cb.reshape(C, hb, W)
    preds_ref[0] = preds.reshape(_PRED_CH, hb, W)


def _pick_rows(h):
    for hb in (_ROWS_PER_STEP, 64, 32, 16, 8):
        if h % hb == 0:
            return hb
    return h


def kernel(imgs, masks, msgs, w_img_t, w_msg, b_h, w_out_t, b_out,
           w_det_t, b_det):
    B, C, H, W = imgs.shape
    hb = _pick_rows(H)
    n_rt = H // hb

    msg_pm1 = 2.0 * msgs.astype(jnp.float32) - 1.0
    mbias = (msg_pm1 @ w_msg + b_h).reshape(B, _HIDDEN, 1)
    wimg_aug = jnp.concatenate(
        [jnp.broadcast_to(w_img_t.astype(jnp.float32)[None],
                          (B, _HIDDEN, C)), mbias], axis=2)

    def band_map(i):
        return (i // n_rt, 0, i % n_rt, 0)

    def batch_map(i):
        return (i // n_rt, 0, 0)

    def weight_map(i):
        return (0, 0)

    in_specs = [
        pl.BlockSpec((1, C, hb, W), band_map),
        pl.BlockSpec((1, 1, hb, W), band_map),
        pl.BlockSpec((1, _HIDDEN, C + 1), batch_map),
        pl.BlockSpec(w_out_t.shape, weight_map),
        pl.BlockSpec(b_out.shape, weight_map),
        pl.BlockSpec(w_det_t.shape, weight_map),
        pl.BlockSpec(b_det.shape, weight_map),
    ]
    out_specs = (
        pl.BlockSpec((1, C, hb, W), band_map),
        pl.BlockSpec((1, C, hb, W), band_map),
        pl.BlockSpec((1, _PRED_CH, hb, W), band_map),
    )
    out_shapes = (
        jax.ShapeDtypeStruct((B, C, H, W), jnp.bfloat16),
        jax.ShapeDtypeStruct((B, C, H, W), jnp.bfloat16),
        jax.ShapeDtypeStruct((B, _PRED_CH, H, W), jnp.bfloat16),
    )

    return pl.pallas_call(
        _wam_kernel,
        out_shape=out_shapes,
        grid_spec=pltpu.PrefetchScalarGridSpec(
            num_scalar_prefetch=0,
            grid=(B * n_rt,),
            in_specs=in_specs,
            out_specs=out_specs),
        compiler_params=pltpu.CompilerParams(
            dimension_semantics=("parallel",)),
    )(imgs, masks, wimg_aug, w_out_t, b_out, w_det_t, b_det)


# speedup vs baseline: 14.2563x; 1.0735x over previous
import jax
import jax.numpy as jnp
from jax import lax
from jax.experimental import pallas as pl
from jax.experimental.pallas import tpu as pltpu

_NBITS = 8
_HIDDEN = 32
_PRED_CH = 1 + _NBITS
_ROWS_PER_STEP = 256


def _wam_kernel(imgs_ref, mask_ref, wimg_ref, wout_ref, bout_ref,
                wdet_ref, bdet_ref, imgs_w_ref, comb_ref, preds_ref):
    C = imgs_ref.shape[1]
    hb = imgs_ref.shape[2]
    W = imgs_ref.shape[3]
    P = hb * W

    wimg = wimg_ref[0]
    wout = wout_ref[...].astype(jnp.bfloat16)
    bout = bout_ref[...]
    wdet = wdet_ref[...].astype(jnp.bfloat16)
    bdet = bdet_ref[...]

    x = imgs_ref[0].reshape(C, P)
    m = mask_ref[0].reshape(1, P)
    xa = jnp.concatenate([x, jnp.ones((1, P), jnp.float32)], axis=0)

    hpre = jnp.dot(wimg, xa, preferred_element_type=jnp.float32)
    h = jnp.tanh(hpre.astype(jnp.bfloat16))
    delta = jnp.tanh(
        jnp.dot(wout, h, preferred_element_type=jnp.float32) + bout)

    iw = (x + delta).astype(jnp.bfloat16)
    cb = (x + m * delta).astype(jnp.bfloat16)
    preds = (jnp.dot(wdet, cb, preferred_element_type=jnp.float32)
             + bdet).astype(jnp.bfloat16)

    imgs_w_ref[0] = iw.reshape(C, hb, W)
    comb_ref[0] = cb.reshape(C, hb, W)
    preds_ref[0] = preds.reshape(_PRED_CH, hb, W)


def _pick_rows(h):
    for hb in (_ROWS_PER_STEP, 128, 64, 32, 16, 8):
        if h % hb == 0:
            return hb
    return h


def kernel(imgs, masks, msgs, w_img_t, w_msg, b_h, w_out_t, b_out,
           w_det_t, b_det):
    B, C, H, W = imgs.shape
    hb = _pick_rows(H)
    n_rt = H // hb

    msg_pm1 = 2.0 * msgs.astype(jnp.float32) - 1.0
    mbias = (msg_pm1 @ w_msg + b_h).reshape(B, _HIDDEN, 1)
    wimg_aug = jnp.concatenate(
        [jnp.broadcast_to(w_img_t.astype(jnp.float32)[None],
                          (B, _HIDDEN, C)), mbias], axis=2)

    def band_map(i):
        return (i // n_rt, 0, i % n_rt, 0)

    def batch_map(i):
        return (i // n_rt, 0, 0)

    def weight_map(i):
        return (0, 0)

    in_specs = [
        pl.BlockSpec((1, C, hb, W), band_map),
        pl.BlockSpec((1, 1, hb, W), band_map),
        pl.BlockSpec((1, _HIDDEN, C + 1), batch_map),
        pl.BlockSpec(w_out_t.shape, weight_map),
        pl.BlockSpec(b_out.shape, weight_map),
        pl.BlockSpec(w_det_t.shape, weight_map),
        pl.BlockSpec(b_det.shape, weight_map),
    ]
    out_specs = (
        pl.BlockSpec((1, C, hb, W), band_map),
        pl.BlockSpec((1, C, hb, W), band_map),
        pl.BlockSpec((1, _PRED_CH, hb, W), band_map),
    )
    out_shapes = (
        jax.ShapeDtypeStruct((B, C, H, W), jnp.bfloat16),
        jax.ShapeDtypeStruct((B, C, H, W), jnp.bfloat16),
        jax.ShapeDtypeStruct((B, _PRED_CH, H, W), jnp.bfloat16),
    )

    return pl.pallas_call(
        _wam_kernel,
        out_shape=out_shapes,
        grid_spec=pltpu.PrefetchScalarGridSpec(
            num_scalar_prefetch=0,
            grid=(B * n_rt,),
            in_specs=in_specs,
            out_specs=out_specs),
        compiler_params=pltpu.CompilerParams(
            dimension_semantics=("parallel",)),
    )(imgs, masks, wimg_aug, w_out_t, b_out, w_det_t, b_det)
